# baseline (speedup 1.0000x reference)
import functools

import jax
import jax.numpy as jnp
from jax import lax
from jax.experimental import pallas as pl
from jax.experimental.pallas import tpu as pltpu
from jax.experimental.pallas import tpu_sc as plsc

_EPS = 1e-5
_NC = 2
_NS = 16
_CHUNK = 64
_EC = 112
_NBUF = 3
_SUP = 24


def _ceil_to(a, m):
    return (a + m - 1) // m * m


def _zero_acc(z_hbm, buf, acc_sp, s, nz):
    pltpu.sync_copy(z_hbm, buf)

    def zstep(k, carry):
        ch = s + k * _NS

        @pl.when(ch < nz)
        def _():
            pltpu.sync_copy(buf, acc_sp.at[pl.ds(ch * _CHUNK, _CHUNK)])
        return carry

    lax.fori_loop(0, -(-nz // _NS), zstep, 0, unroll=True)


def _copy_out(acc_sp, buf, out_hbm, c, s, npc):
    def ostep(k, carry):
        ch = s + k * _NS

        @pl.when(ch < npc)
        def _():
            pltpu.sync_copy(acc_sp.at[pl.ds(ch * _CHUNK, _CHUNK)], buf)
            pltpu.sync_copy(buf, out_hbm.at[c, pl.ds(ch * _CHUNK, _CHUNK)])
        return carry

    lax.fori_loop(0, -(-npc // _NS), ostep, 0, unroll=True)


@functools.lru_cache(maxsize=None)
def _make_seg_sum(n_pad, d, n_chunks):
    acc_rows = n_pad
    nz = acc_rows // _CHUNK
    npc = n_pad // _CHUNK
    mesh = plsc.VectorSubcoreMesh(core_axis_name="c", subcore_axis_name="s")

    out_type = jax.ShapeDtypeStruct((_NC, n_pad, d), jnp.float32)
    scratch = (
        [pltpu.VMEM((_SUP, _EC), jnp.int32)] * 2
        + [pltpu.VMEM((_EC, d), jnp.float32)] * _NBUF
        + [pltpu.VMEM_SHARED((acc_rows, d), jnp.float32)]
        + [pltpu.SemaphoreType.DMA] * _NBUF
    )

    def body(tab_hbm, src_hbm, dst_hbm, z_hbm, out_hbm,
             sidx, didx, *rest):
        bufs = rest[:_NBUF]
        acc_sp = rest[_NBUF]
        sems = rest[_NBUF + 1:]
        s = lax.axis_index("s")
        c = lax.axis_index("c")

        stage = bufs[0].at[pl.ds(0, _CHUNK)]
        _zero_acc(z_hbm, stage, acc_sp, s, nz)
        plsc.subcore_barrier()

        def sup_step(u, carry):
            pltpu.sync_copy(src_hbm.at[c, s, pl.ds(u * _SUP, _SUP)], sidx)
            pltpu.sync_copy(dst_hbm.at[c, s, pl.ds(u * _SUP, _SUP)], didx)

            def step(t, carry2):
                j = t * _NBUF
                cps = [pltpu.async_copy(tab_hbm.at[sidx.at[j + b]], bufs[b],
                                        sems[b]) for b in range(_NBUF)]
                for b in range(_NBUF):
                    cps[b].wait()
                    pltpu.sync_copy(bufs[b], acc_sp.at[didx.at[j + b]],
                                    add=True)
                return carry2

            lax.fori_loop(0, _SUP // _NBUF, step, 0)
            return carry

        lax.fori_loop(0, n_chunks // _SUP, sup_step, 0)
        plsc.subcore_barrier()

        _copy_out(acc_sp, bufs[0].at[pl.ds(0, _CHUNK)], out_hbm, c, s, npc)

    return pl.kernel(body, out_type=out_type, mesh=mesh,
                     scratch_types=scratch)


@functools.lru_cache(maxsize=None)
def _make_degree(n_pad, d, n_chunks):
    acc_rows = n_pad
    nz = acc_rows // _CHUNK
    npc = n_pad // _CHUNK
    mesh = plsc.VectorSubcoreMesh(core_axis_name="c", subcore_axis_name="s")

    out_type = jax.ShapeDtypeStruct((_NC, n_pad, d), jnp.float32)
    scratch = [
        pltpu.VMEM((n_chunks, _EC), jnp.int32),
        pltpu.VMEM((_EC, d), jnp.float32),
        pltpu.VMEM((_CHUNK, d), jnp.float32),
        pltpu.VMEM_SHARED((acc_rows, d), jnp.float32),
    ]

    def body(one_hbm, dst_hbm, z_hbm, out_hbm, dst_v, ones_v, stage_v, acc_sp):
        s = lax.axis_index("s")
        c = lax.axis_index("c")

        _zero_acc(z_hbm, stage_v, acc_sp, s, nz)
        pltpu.sync_copy(one_hbm, ones_v)
        pltpu.sync_copy(dst_hbm.at[c, s], dst_v)
        plsc.subcore_barrier()

        def step(i, carry):
            pltpu.sync_copy(ones_v, acc_sp.at[dst_v.at[i]], add=True)
            return carry

        lax.fori_loop(0, n_chunks, step, 0)
        plsc.subcore_barrier()

        _copy_out(acc_sp, stage_v, out_hbm, c, s, npc)

    return pl.kernel(body, out_type=out_type, mesh=mesh,
                     scratch_types=scratch)


def _row_mask(shape, n_valid):
    return lax.broadcasted_iota(jnp.int32, shape, 0) < n_valid


def _bn_relu(y, g, b, n_valid, relu=True):
    mask = _row_mask(y.shape, n_valid)
    ym = jnp.where(mask, y, 0.0)
    m = jnp.sum(ym, axis=0, keepdims=True) * (1.0 / n_valid)
    yc = jnp.where(mask, y - m, 0.0)
    v = jnp.sum(yc * yc, axis=0, keepdims=True) * (1.0 / n_valid)
    o = yc * (g * lax.rsqrt(v + _EPS)) + b
    if relu:
        o = jnp.maximum(o, 0.0)
    return jnp.where(mask, o, 0.0)


def _self_mm():
    def f(h_ref, w_ref, o_ref):
        o_ref[...] = jnp.dot(h_ref[...], w_ref[...],
                             preferred_element_type=jnp.float32)
    return f


def _sage_mm(n_valid, bn):
    def f(hw_ref, a0_ref, a1_ref, d0_ref, d1_ref, wl_ref,
          g_ref, b_ref, o_ref):
        inv = 1.0 / jnp.maximum(d0_ref[...] + d1_ref[...], 1.0)
        mean = (a0_ref[...] + a1_ref[...]) * inv
        y = jnp.dot(mean, wl_ref[...], preferred_element_type=jnp.float32)
        y = y + hw_ref[...]
        if bn:
            o_ref[...] = _bn_relu(y, g_ref[...], b_ref[...], n_valid)
        else:
            mask = _row_mask(y.shape, n_valid)
            o_ref[...] = jnp.where(mask, y + b_ref[...], 0.0)
    return f


def _lin_bn(n_valid):
    def f(h_ref, w_ref, g_ref, b_ref, o_ref):
        y = jnp.dot(h_ref[...], w_ref[...], preferred_element_type=jnp.float32)
        o_ref[...] = _bn_relu(y, g_ref[...], b_ref[...], n_valid)
    return f


def _tr2_dec0(n_valid, rb):
    def f(h_ref, w2_ref, b2_ref, w0_ref, mu_ref, t_ref):
        i = pl.program_id(0)
        z = jnp.dot(h_ref[...], w2_ref[...],
                    preferred_element_type=jnp.float32) + b2_ref[...]
        mu_ref[...] = z
        mask = (lax.broadcasted_iota(jnp.int32, z.shape, 0) + i * rb) < n_valid
        zm = jnp.where(mask, z, 0.0)
        t_ref[...] = jnp.dot(zm, w0_ref[...],
                             preferred_element_type=jnp.float32)
    return f


def _dec_head(n_valid):
    def f(t_ref, g5_ref, b5_ref, w1_ref, g4_ref, b4_ref, o_ref):
        a = _bn_relu(t_ref[...], g5_ref[...], b5_ref[...], n_valid)
        y = jnp.dot(a, w1_ref[...], preferred_element_type=jnp.float32)
        o_ref[...] = _bn_relu(y, g4_ref[...], b4_ref[...], n_valid)
    return f


def kernel(x, adj, lengs, size, s1_wl, s1_wr, s1_b, s2_wl, s2_wr, s2_b,
           s3_wl, s3_wr, s3_b, s4_wl, s4_wr, s4_b, tr1_w, tr1_b, tr2_w,
           tr2_b, dec0_w, dec0_b, dec1_w, dec1_b, d2_wl, d2_wr, d2_b,
           d3_wl, d3_wr, d3_b, d4_wl, d4_wr, d4_b, d5_wl, d5_wr, d5_b,
           bn1_g, bn1_b, bn2_g, bn2_b, bn3_g, bn3_b, bn4_g, bn4_b,
           bn5_g, bn5_b):
    f32 = jnp.float32
    n, d = x.shape
    e = adj.shape[1]
    n_pad = _ceil_to(n + 1, _CHUNK)
    e_half = -(-e // _NC)
    n_chunks = _ceil_to(-(-e_half // (_NS * _EC)), _SUP)
    e_sc = _NS * n_chunks * _EC

    pad_idx = jnp.full((_NC * e_sc - e,), n, jnp.int32)
    src = jnp.concatenate([adj[0], pad_idx]).reshape(_NC, _NS, n_chunks,
                                                     _EC)
    dst = jnp.concatenate([adj[1], pad_idx]).reshape(_NC, _NS, n_chunks,
                                                     _EC)
    zeros_blk = jnp.zeros((_CHUNK, d), f32)
    ones_blk = jnp.ones((_EC, d), f32)

    x_p = jnp.zeros((n_pad, d), f32).at[:n].set(x)

    seg_sum = _make_seg_sum(n_pad, d, n_chunks)
    degree = _make_degree(n_pad, d, n_chunks)

    def seg(h):
        return seg_sum(h, src, dst, zeros_blk)

    def r2(v):
        return v.reshape(1, -1)

    def sage(h, wl, wr, b, g, beta, bn=True):
        hw = pl.pallas_call(
            _self_mm(),
            out_shape=jax.ShapeDtypeStruct((n_pad, d), f32),
        )(h, wr.T)
        agg = seg(h)
        gb = (r2(g), r2(beta)) if bn else (r2(b), r2(b))
        return pl.pallas_call(
            _sage_mm(n, bn),
            out_shape=jax.ShapeDtypeStruct((n_pad, d), f32),
        )(hw, agg[0], agg[1], deg0, deg1, wl.T, *gb)

    degs = degree(ones_blk, dst, zeros_blk)
    deg0 = degs[0, :, 0:1]
    deg1 = degs[1, :, 0:1]

    h = sage(x_p, s1_wl, s1_wr, s1_b, bn1_g, bn1_b)
    h = sage(h, s2_wl, s2_wr, s2_b, bn2_g, bn2_b)
    h = sage(h, s3_wl, s3_wr, s3_b, bn3_g, bn3_b)
    h = sage(h, s4_wl, s4_wr, s4_b, bn4_g, bn4_b)

    k1 = tr1_w.shape[0]
    h = pl.pallas_call(
        _lin_bn(n),
        out_shape=jax.ShapeDtypeStruct((n_pad, k1), f32),
    )(h, tr1_w.T, r2(bn5_g), r2(bn5_b))

    k2 = tr2_w.shape[0]
    nb = 8
    rb = n_pad // nb
    mu_full, t = pl.pallas_call(
        _tr2_dec0(n, rb),
        grid=(nb,),
        in_specs=[
            pl.BlockSpec((rb, k1), lambda i: (i, 0)),
            pl.BlockSpec((k1, k2), lambda i: (0, 0)),
            pl.BlockSpec((1, k2), lambda i: (0, 0)),
            pl.BlockSpec((k2, k1), lambda i: (0, 0)),
        ],
        out_specs=[
            pl.BlockSpec((rb, k2), lambda i: (i, 0)),
            pl.BlockSpec((rb, k1), lambda i: (i, 0)),
        ],
        out_shape=[jax.ShapeDtypeStruct((n_pad, k2), f32),
                   jax.ShapeDtypeStruct((n_pad, k1), f32)],
    )(h, tr2_w.T, r2(tr2_b), dec0_w.T)

    o = pl.pallas_call(
        _dec_head(n),
        out_shape=jax.ShapeDtypeStruct((n_pad, d), f32),
    )(t, r2(bn5_g), r2(bn5_b), dec1_w.T, r2(bn4_g), r2(bn4_b))

    o = sage(o, d2_wl, d2_wr, d2_b, bn3_g, bn3_b)
    o = sage(o, d3_wl, d3_wr, d3_b, bn2_g, bn2_b)
    o = sage(o, d4_wl, d4_wr, d4_b, bn1_g, bn1_b)
    z2 = sage(o, d5_wl, d5_wr, d5_b, None, None, bn=False)

    return z2[:n], mu_full[:n], mu_full[:n]


# speedup vs baseline: 2.1994x; 2.1994x over previous
import functools

import jax
import jax.numpy as jnp
from jax import lax
from jax.experimental import pallas as pl
from jax.experimental.pallas import tpu as pltpu
from jax.experimental.pallas import tpu_sc as plsc

_EPS = 1e-5
_NC = 2
_NS = 16
_CHUNK = 64
_EC = 128
_NBUF = 2
_SUP = 16


def _ceil_to(a, m):
    return (a + m - 1) // m * m


def _zero_acc(z_hbm, buf, acc_sp, s, nz):
    pltpu.sync_copy(z_hbm, buf)

    def zstep(k, carry):
        ch = s + k * _NS

        @pl.when(ch < nz)
        def _():
            pltpu.sync_copy(buf, acc_sp.at[pl.ds(ch * _CHUNK, _CHUNK)])
        return carry

    lax.fori_loop(0, -(-nz // _NS), zstep, 0, unroll=True)


def _copy_out(acc_sp, buf, out_hbm, c, s, npc):
    def ostep(k, carry):
        ch = s + k * _NS

        @pl.when(ch < npc)
        def _():
            pltpu.sync_copy(acc_sp.at[pl.ds(ch * _CHUNK, _CHUNK)], buf)
            pltpu.sync_copy(buf, out_hbm.at[c, pl.ds(ch * _CHUNK, _CHUNK)])
        return carry

    lax.fori_loop(0, -(-npc // _NS), ostep, 0, unroll=True)


@functools.lru_cache(maxsize=None)
def _make_seg_sum(n_pad, d, n_chunks):
    acc_rows = n_pad
    nz = acc_rows // _CHUNK
    npc = n_pad // _CHUNK
    mesh = plsc.VectorSubcoreMesh(core_axis_name="c", subcore_axis_name="s")

    out_type = jax.ShapeDtypeStruct((_NC, n_pad, d), jnp.float32)
    scratch = (
        [pltpu.VMEM((_SUP, _EC), jnp.int32)] * 2
        + [pltpu.VMEM((_EC, d), jnp.float32)] * _NBUF
        + [pltpu.VMEM_SHARED((acc_rows, d), jnp.float32)]
        + [pltpu.SemaphoreType.DMA] * _NBUF
    )

    def body(tab_hbm, src_hbm, dst_hbm, z_hbm, out_hbm,
             sidx, didx, *rest):
        bufs = rest[:_NBUF]
        acc_sp = rest[_NBUF]
        sems = rest[_NBUF + 1:]
        s = lax.axis_index("s")
        c = lax.axis_index("c")

        stage = bufs[0].at[pl.ds(0, _CHUNK)]
        _zero_acc(z_hbm, stage, acc_sp, s, nz)
        plsc.subcore_barrier()

        def sup_step(u, carry):
            pltpu.sync_copy(src_hbm.at[c, s, pl.ds(u * _SUP, _SUP)], sidx)
            pltpu.sync_copy(dst_hbm.at[c, s, pl.ds(u * _SUP, _SUP)], didx)

            def step(t, carry2):
                j = t * _NBUF
                cps = [pltpu.async_copy(tab_hbm.at[sidx.at[j + b]], bufs[b],
                                        sems[b]) for b in range(_NBUF)]
                for b in range(_NBUF):
                    cps[b].wait()
                    pltpu.sync_copy(bufs[b], acc_sp.at[didx.at[j + b]],
                                    add=True)
                return carry2

            lax.fori_loop(0, _SUP // _NBUF, step, 0)
            return carry

        lax.fori_loop(0, n_chunks // _SUP, sup_step, 0)
        plsc.subcore_barrier()

        _copy_out(acc_sp, bufs[0].at[pl.ds(0, _CHUNK)], out_hbm, c, s, npc)

    return pl.kernel(body, out_type=out_type, mesh=mesh,
                     scratch_types=scratch)


@functools.lru_cache(maxsize=None)
def _make_degree(n_pad, d, n_chunks):
    acc_rows = n_pad
    nz = acc_rows // _CHUNK
    npc = n_pad // _CHUNK
    mesh = plsc.VectorSubcoreMesh(core_axis_name="c", subcore_axis_name="s")

    out_type = jax.ShapeDtypeStruct((_NC, n_pad, d), jnp.float32)
    scratch = [
        pltpu.VMEM((n_chunks, _EC), jnp.int32),
        pltpu.VMEM((_EC, d), jnp.float32),
        pltpu.VMEM((_CHUNK, d), jnp.float32),
        pltpu.VMEM_SHARED((acc_rows, d), jnp.float32),
    ]

    def body(one_hbm, dst_hbm, z_hbm, out_hbm, dst_v, ones_v, stage_v, acc_sp):
        s = lax.axis_index("s")
        c = lax.axis_index("c")

        _zero_acc(z_hbm, stage_v, acc_sp, s, nz)
        pltpu.sync_copy(one_hbm, ones_v)
        pltpu.sync_copy(dst_hbm.at[c, s], dst_v)
        plsc.subcore_barrier()

        def step(i, carry):
            pltpu.sync_copy(ones_v, acc_sp.at[dst_v.at[i]], add=True)
            return carry

        lax.fori_loop(0, n_chunks, step, 0)
        plsc.subcore_barrier()

        _copy_out(acc_sp, stage_v, out_hbm, c, s, npc)

    return pl.kernel(body, out_type=out_type, mesh=mesh,
                     scratch_types=scratch)


def _row_mask(shape, n_valid):
    return lax.broadcasted_iota(jnp.int32, shape, 0) < n_valid


def _bn_relu(y, g, b, n_valid, relu=True):
    mask = _row_mask(y.shape, n_valid)
    ym = jnp.where(mask, y, 0.0)
    m = jnp.sum(ym, axis=0, keepdims=True) * (1.0 / n_valid)
    yc = jnp.where(mask, y - m, 0.0)
    v = jnp.sum(yc * yc, axis=0, keepdims=True) * (1.0 / n_valid)
    o = yc * (g * lax.rsqrt(v + _EPS)) + b
    if relu:
        o = jnp.maximum(o, 0.0)
    return jnp.where(mask, o, 0.0)


def _self_mm():
    def f(h_ref, w_ref, o_ref):
        o_ref[...] = jnp.dot(h_ref[...], w_ref[...],
                             preferred_element_type=jnp.float32)
    return f


def _sage_mm(n_valid, bn):
    def f(hw_ref, a0_ref, a1_ref, d0_ref, d1_ref, wl_ref,
          g_ref, b_ref, o_ref):
        inv = 1.0 / jnp.maximum(d0_ref[...] + d1_ref[...], 1.0)
        mean = (a0_ref[...] + a1_ref[...]) * inv
        y = jnp.dot(mean, wl_ref[...], preferred_element_type=jnp.float32)
        y = y + hw_ref[...]
        if bn:
            o_ref[...] = _bn_relu(y, g_ref[...], b_ref[...], n_valid)
        else:
            mask = _row_mask(y.shape, n_valid)
            o_ref[...] = jnp.where(mask, y + b_ref[...], 0.0)
    return f


def _lin_bn(n_valid):
    def f(h_ref, w_ref, g_ref, b_ref, o_ref):
        y = jnp.dot(h_ref[...], w_ref[...], preferred_element_type=jnp.float32)
        o_ref[...] = _bn_relu(y, g_ref[...], b_ref[...], n_valid)
    return f


def _tr2_dec0(n_valid, rb):
    def f(h_ref, w2_ref, b2_ref, w0_ref, mu_ref, t_ref):
        i = pl.program_id(0)
        z = jnp.dot(h_ref[...], w2_ref[...],
                    preferred_element_type=jnp.float32) + b2_ref[...]
        mu_ref[...] = z
        mask = (lax.broadcasted_iota(jnp.int32, z.shape, 0) + i * rb) < n_valid
        zm = jnp.where(mask, z, 0.0)
        t_ref[...] = jnp.dot(zm, w0_ref[...],
                             preferred_element_type=jnp.float32)
    return f


def _dec_head(n_valid):
    def f(t_ref, g5_ref, b5_ref, w1_ref, g4_ref, b4_ref, o_ref):
        a = _bn_relu(t_ref[...], g5_ref[...], b5_ref[...], n_valid)
        y = jnp.dot(a, w1_ref[...], preferred_element_type=jnp.float32)
        o_ref[...] = _bn_relu(y, g4_ref[...], b4_ref[...], n_valid)
    return f


def kernel(x, adj, lengs, size, s1_wl, s1_wr, s1_b, s2_wl, s2_wr, s2_b,
           s3_wl, s3_wr, s3_b, s4_wl, s4_wr, s4_b, tr1_w, tr1_b, tr2_w,
           tr2_b, dec0_w, dec0_b, dec1_w, dec1_b, d2_wl, d2_wr, d2_b,
           d3_wl, d3_wr, d3_b, d4_wl, d4_wr, d4_b, d5_wl, d5_wr, d5_b,
           bn1_g, bn1_b, bn2_g, bn2_b, bn3_g, bn3_b, bn4_g, bn4_b,
           bn5_g, bn5_b):
    f32 = jnp.float32
    n, d = x.shape
    e = adj.shape[1]
    n_pad = _ceil_to(n + 1, _CHUNK)
    e_half = -(-e // _NC)
    n_chunks = _ceil_to(-(-e_half // (_NS * _EC)), _SUP)
    e_sc = _NS * n_chunks * _EC

    pad_idx = jnp.full((_NC * e_sc - e,), n, jnp.int32)
    src = jnp.concatenate([adj[0], pad_idx]).reshape(_NC, _NS, n_chunks,
                                                     _EC)
    dst = jnp.concatenate([adj[1], pad_idx]).reshape(_NC, _NS, n_chunks,
                                                     _EC)
    zeros_blk = jnp.zeros((_CHUNK, d), f32)
    ones_blk = jnp.ones((_EC, d), f32)

    x_p = jnp.zeros((n_pad, d), f32).at[:n].set(x)

    seg_sum = _make_seg_sum(n_pad, d, n_chunks)
    degree = _make_degree(n_pad, d, n_chunks)

    def seg(h):
        return seg_sum(h, src, dst, zeros_blk)

    def r2(v):
        return v.reshape(1, -1)

    def sage(h, wl, wr, b, g, beta, bn=True):
        hw = pl.pallas_call(
            _self_mm(),
            out_shape=jax.ShapeDtypeStruct((n_pad, d), f32),
        )(h, wr.T)
        agg = seg(h)
        gb = (r2(g), r2(beta)) if bn else (r2(b), r2(b))
        return pl.pallas_call(
            _sage_mm(n, bn),
            out_shape=jax.ShapeDtypeStruct((n_pad, d), f32),
        )(hw, agg[0], agg[1], deg0, deg1, wl.T, *gb)

    degs = degree(ones_blk, dst, zeros_blk)
    deg0 = degs[0, :, 0:1]
    deg1 = degs[1, :, 0:1]

    h = sage(x_p, s1_wl, s1_wr, s1_b, bn1_g, bn1_b)
    h = sage(h, s2_wl, s2_wr, s2_b, bn2_g, bn2_b)
    h = sage(h, s3_wl, s3_wr, s3_b, bn3_g, bn3_b)
    h = sage(h, s4_wl, s4_wr, s4_b, bn4_g, bn4_b)

    k1 = tr1_w.shape[0]
    h = pl.pallas_call(
        _lin_bn(n),
        out_shape=jax.ShapeDtypeStruct((n_pad, k1), f32),
    )(h, tr1_w.T, r2(bn5_g), r2(bn5_b))

    k2 = tr2_w.shape[0]
    nb = 8
    rb = n_pad // nb
    mu_full, t = pl.pallas_call(
        _tr2_dec0(n, rb),
        grid=(nb,),
        in_specs=[
            pl.BlockSpec((rb, k1), lambda i: (i, 0)),
            pl.BlockSpec((k1, k2), lambda i: (0, 0)),
            pl.BlockSpec((1, k2), lambda i: (0, 0)),
            pl.BlockSpec((k2, k1), lambda i: (0, 0)),
        ],
        out_specs=[
            pl.BlockSpec((rb, k2), lambda i: (i, 0)),
            pl.BlockSpec((rb, k1), lambda i: (i, 0)),
        ],
        out_shape=[jax.ShapeDtypeStruct((n_pad, k2), f32),
                   jax.ShapeDtypeStruct((n_pad, k1), f32)],
    )(h, tr2_w.T, r2(tr2_b), dec0_w.T)

    o = pl.pallas_call(
        _dec_head(n),
        out_shape=jax.ShapeDtypeStruct((n_pad, d), f32),
    )(t, r2(bn5_g), r2(bn5_b), dec1_w.T, r2(bn4_g), r2(bn4_b))

    o = sage(o, d2_wl, d2_wr, d2_b, bn3_g, bn3_b)
    o = sage(o, d3_wl, d3_wr, d3_b, bn2_g, bn2_b)
    o = sage(o, d4_wl, d4_wr, d4_b, bn1_g, bn1_b)
    z2 = sage(o, d5_wl, d5_wr, d5_b, None, None, bn=False)

    return z2[:n], mu_full[:n], mu_full[:n]
